# baseline (speedup 1.0000x reference)
import jax
import jax.numpy as jnp
from jax.experimental import pallas as pl

N_ROWS = 16384
N_CLASSES = 1000
BLOCK_R = 4096
GAMMA = 2.0
EPS = 1e-5


def _hist_kernel(t_ref, hist_ref):
    i = pl.program_id(0)

    @pl.when(i == 0)
    def _():
        hist_ref[...] = jnp.zeros_like(hist_ref)

    t = t_ref[...]
    iota = jax.lax.broadcasted_iota(jnp.int32, (BLOCK_R, N_CLASSES), 1)
    mask = iota == t
    hist_ref[...] += jnp.sum(mask.astype(jnp.float32), axis=0, keepdims=True)


def _main_kernel(x_ref, t_ref, hist_ref, out_ref):
    i = pl.program_id(0)

    @pl.when(i == 0)
    def _():
        out_ref[...] = jnp.zeros_like(out_ref)

    hist = hist_ref[...]
    freq = hist / jnp.sum(hist)
    alpha_raw = 1.0 / (freq + EPS)
    alpha = alpha_raw / jnp.sum(alpha_raw)

    x = x_ref[...]
    t = t_ref[...]
    m = jnp.max(x, axis=1, keepdims=True)
    e = jnp.exp(x - m)
    iota = jax.lax.broadcasted_iota(jnp.int32, (BLOCK_R, N_CLASSES), 1)
    mask = iota == t
    w = jnp.where(mask, x, 0.0)
    aw = jnp.where(mask, jnp.broadcast_to(alpha, x.shape), 0.0)
    ones = jnp.ones((N_CLASSES, 1), jnp.float32)
    s = jax.lax.dot_general(e, ones, (((1,), (0,)), ((), ())),
                            preferred_element_type=jnp.float32)
    xt = jax.lax.dot_general(w, ones, (((1,), (0,)), ((), ())),
                             preferred_element_type=jnp.float32)
    a = jax.lax.dot_general(aw, ones, (((1,), (0,)), ((), ())),
                            preferred_element_type=jnp.float32)

    nll = m + jnp.log(s) - xt
    ce = a * nll
    pt = jnp.exp(-ce)
    contrib = (1.0 - pt) ** GAMMA * ce
    out_ref[...] += jnp.sum(contrib).reshape(1, 1) / N_ROWS


def kernel(inputs, targets):
    targets = targets.astype(jnp.int32).reshape(N_ROWS, 1)
    nb = N_ROWS // BLOCK_R

    hist = pl.pallas_call(
        _hist_kernel,
        grid=(nb,),
        in_specs=[pl.BlockSpec((BLOCK_R, 1), lambda i: (i, 0))],
        out_specs=pl.BlockSpec((1, N_CLASSES), lambda i: (0, 0)),
        out_shape=jax.ShapeDtypeStruct((1, N_CLASSES), jnp.float32),
    )(targets)

    out = pl.pallas_call(
        _main_kernel,
        grid=(nb,),
        in_specs=[
            pl.BlockSpec((BLOCK_R, N_CLASSES), lambda i: (i, 0)),
            pl.BlockSpec((BLOCK_R, 1), lambda i: (i, 0)),
            pl.BlockSpec((1, N_CLASSES), lambda i: (0, 0)),
        ],
        out_specs=pl.BlockSpec((1, 1), lambda i: (0, 0)),
        out_shape=jax.ShapeDtypeStruct((1, 1), jnp.float32),
    )(inputs, targets, hist)

    return out[0, 0]


# speedup vs baseline: 1.0301x; 1.0301x over previous
import jax
import jax.numpy as jnp
from jax.experimental import pallas as pl
from jax.experimental.pallas import tpu as pltpu

N_ROWS = 16384
N_CLASSES = 1000
BLOCK_R = 2048
GAMMA = 2.0
EPS = 1e-5


def _hist_kernel(t_ref, hist_ref):
    i = pl.program_id(0)

    @pl.when(i == 0)
    def _():
        hist_ref[...] = jnp.zeros_like(hist_ref)

    t = t_ref[...]
    iota = jax.lax.broadcasted_iota(jnp.int32, (BLOCK_R, N_CLASSES), 1)
    mask = iota == t
    hist_ref[...] += jnp.sum(mask.astype(jnp.float32), axis=0, keepdims=True)


def _main_kernel(x_ref, t_ref, hist_ref, out_ref):
    hist = hist_ref[...]
    freq = hist / jnp.sum(hist)
    alpha_raw = 1.0 / (freq + EPS)
    alpha = alpha_raw / jnp.sum(alpha_raw)

    x = x_ref[...]
    t = t_ref[...]
    m = jnp.max(x, axis=1, keepdims=True)
    e = jnp.exp(x - m)
    iota = jax.lax.broadcasted_iota(jnp.int32, (BLOCK_R, N_CLASSES), 1)
    mask = iota == t
    w = jnp.where(mask, x, 0.0)
    aw = jnp.where(mask, jnp.broadcast_to(alpha, x.shape), 0.0)
    ones = jnp.ones((N_CLASSES, 1), jnp.float32)
    s = jax.lax.dot_general(e, ones, (((1,), (0,)), ((), ())),
                            preferred_element_type=jnp.float32)
    xt = jax.lax.dot_general(w, ones, (((1,), (0,)), ((), ())),
                             preferred_element_type=jnp.float32)
    a = jax.lax.dot_general(aw, ones, (((1,), (0,)), ((), ())),
                            preferred_element_type=jnp.float32)

    nll = m + jnp.log(s) - xt
    ce = a * nll
    pt = jnp.exp(-ce)
    contrib = (1.0 - pt) ** GAMMA * ce
    out_ref[...] = jnp.sum(contrib).reshape(1, 1, 1) / N_ROWS


def kernel(inputs, targets):
    targets = targets.astype(jnp.int32).reshape(N_ROWS, 1)
    nb = N_ROWS // BLOCK_R

    hist = pl.pallas_call(
        _hist_kernel,
        grid=(nb,),
        in_specs=[pl.BlockSpec((BLOCK_R, 1), lambda i: (i, 0))],
        out_specs=pl.BlockSpec((1, N_CLASSES), lambda i: (0, 0)),
        out_shape=jax.ShapeDtypeStruct((1, N_CLASSES), jnp.float32),
    )(targets)

    out = pl.pallas_call(
        _main_kernel,
        grid=(nb,),
        in_specs=[
            pl.BlockSpec((BLOCK_R, N_CLASSES), lambda i: (i, 0)),
            pl.BlockSpec((BLOCK_R, 1), lambda i: (i, 0)),
            pl.BlockSpec((1, N_CLASSES), lambda i: (0, 0)),
        ],
        out_specs=pl.BlockSpec((1, 1, 1), lambda i: (i, 0, 0)),
        out_shape=jax.ShapeDtypeStruct((nb, 1, 1), jnp.float32),
        compiler_params=pltpu.CompilerParams(
            dimension_semantics=("parallel",)),
    )(inputs, targets, hist)

    return jnp.sum(out)
